# baseline (speedup 1.0000x reference)
import functools

import jax
import jax.numpy as jnp
from jax import lax
from jax.experimental import pallas as pl
from jax.experimental.pallas import tpu as pltpu
from jax.experimental.pallas import tpu_sc as plsc

N = 320000
B = 10000
D = 128
L = 16
NV = D // L
NC = 2
NS = 16
NW = NC * NS
SPW = 313
SPWP = 320
BP = NW * SPW
RS_PAD = 10368
K = 160
NEG = -3.0e38

_mesh = plsc.VectorSubcoreMesh(core_axis_name="c", subcore_axis_name="s")


@functools.partial(
    pl.kernel,
    out_type=jax.ShapeDtypeStruct((NW * SPW * D,), jnp.float32),
    mesh=_mesh,
    compiler_params=pltpu.CompilerParams(needs_layout_passes=False),
    scratch_types=[
        pltpu.VMEM((352,), jnp.int32),
        pltpu.VMEM((2, K, D), jnp.float32),
        pltpu.VMEM((K + L,), jnp.int32),
        pltpu.VMEM((K + L,), jnp.int32),
        pltpu.VMEM((SPWP, D), jnp.float32),
        pltpu.VMEM((SPW * D,), jnp.float32),
        pltpu.SemaphoreType.DMA,
        pltpu.SemaphoreType.DMA,
        pltpu.SemaphoreType.DMA,
        pltpu.SemaphoreType.DMA,
    ],
)
def _wg(atom_hbm, ids_hbm, prot_hbm, rs_hbm, out_hbm,
        rsb, ablk, iblk0, iblk1, protb, outb, semA0, semA1, semI0, semI1):
    w = lax.axis_index("s") * NC + lax.axis_index("c")
    s_lo = w * SPW
    rbase = pl.multiple_of(s_lo - lax.rem(s_lo, 8), 8)
    pltpu.sync_copy(rs_hbm.at[pl.ds(rbase, 352)], rsb)
    pltpu.sync_copy(prot_hbm.at[w], protb)

    zvec = jnp.zeros((L,), jnp.float32)
    nvec = jnp.full((L,), NEG, jnp.float32)

    def _zero(r, c):
        outb[pl.ds(r * L, L)] = zvec
        return c

    lax.fori_loop(0, SPW * D // L, _zero, 0)

    a0 = rsb[pl.ds(s_lo - rbase, L)][0]
    a1 = rsb[pl.ds(s_lo - rbase + SPW, L)][0]
    base = pl.multiple_of(a0 - lax.rem(a0, 8), 8)
    nblk = lax.div(a1 - base + (K - 1), K)

    def _offs(b):
        return pl.multiple_of(jnp.minimum(base + b * K, N - K), 8)

    def _start(b, par):
        off = _offs(b)
        if par == 0:
            pltpu.async_copy(atom_hbm.at[pl.ds(off, K), :], ablk.at[0], semA0)
            pltpu.async_copy(ids_hbm.at[pl.ds(off, K)],
                             iblk0.at[pl.ds(0, K)], semI0)
        else:
            pltpu.async_copy(atom_hbm.at[pl.ds(off, K), :], ablk.at[1], semA1)
            pltpu.async_copy(ids_hbm.at[pl.ds(off, K)],
                             iblk1.at[pl.ds(0, K)], semI1)

    def _wait(par):
        if par == 0:
            pltpu.make_async_copy(atom_hbm.at[pl.ds(0, K), :], ablk.at[0],
                                  semA0).wait()
            pltpu.make_async_copy(ids_hbm.at[pl.ds(0, K)],
                                  iblk0.at[pl.ds(0, K)], semI0).wait()
        else:
            pltpu.make_async_copy(atom_hbm.at[pl.ds(0, K), :], ablk.at[1],
                                  semA1).wait()
            pltpu.make_async_copy(ids_hbm.at[pl.ds(0, K)],
                                  iblk1.at[pl.ds(0, K)], semI1).wait()

    @pl.when(nblk > 0)
    def _():
        _start(0, 0)

    def blk_body(b, carry):
        par = lax.rem(b, 2)
        off = _offs(b)

        @pl.when(par == 0)
        def _():
            _wait(0)

        @pl.when(par == 1)
        def _():
            _wait(1)

        nxt = b + 1

        @pl.when(jnp.logical_and(nxt < nblk, lax.rem(nxt, 2) == 0))
        def _():
            _start(nxt, 0)

        @pl.when(jnp.logical_and(nxt < nblk, lax.rem(nxt, 2) == 1))
        def _():
            _start(nxt, 1)

        lo = jnp.maximum(a0, base + b * K) - off
        hi = jnp.minimum(a1, base + (b + 1) * K) - off
        valid = hi > lo
        sf0 = iblk0[pl.ds(lo, L)][0]
        sf1 = iblk1[pl.ds(lo, L)][0]
        sl0 = iblk0[pl.ds(jnp.maximum(hi - 1, 0), L)][0]
        sl1 = iblk1[pl.ds(jnp.maximum(hi - 1, 0), L)][0]
        sf_raw = jnp.where(par == 0, sf0, sf1)
        sl_raw = jnp.where(par == 0, sl0, sl1)
        sfirst = jnp.where(valid, sf_raw, 1)
        slast = jnp.where(valid, sl_raw, 0)

        def seg_body(sv, c):
            m, s, acc = c
            rvec = rsb[pl.ds(sv - rbase, L)]
            r0 = rvec[0]
            r1 = rvec[1]
            fresh = r0 >= off + lo
            m = jnp.where(fresh, nvec, m)
            s = jnp.where(fresh, zvec, s)
            acc = tuple(jnp.where(fresh, zvec, acc[j]) for j in range(NV))
            r = sv - s_lo
            p = tuple(protb[r, pl.ds(j * L, L)] for j in range(NV))
            ilo = jnp.maximum(lo, r0 - off)
            ihi = jnp.minimum(hi, r1 - off)

            def atom_body(i, c2):
                m, s, acc = c2
                a = tuple(ablk[par, i, pl.ds(j * L, L)] for j in range(NV))
                t0 = a[0] * p[0] + a[1] * p[1]
                t1 = a[2] * p[2] + a[3] * p[3]
                t2 = a[4] * p[4] + a[5] * p[5]
                t3 = a[6] * p[6] + a[7] * p[7]
                part = (t0 + t1) + (t2 + t3)
                lv = jnp.full((L,), jnp.sum(part), jnp.float32)
                mn = jnp.maximum(m, lv)
                sc = jnp.exp(m - mn)
                e = jnp.exp(lv - mn)
                s = s * sc + e
                acc = tuple(acc[j] * sc + a[j] * e for j in range(NV))
                return (mn, s, acc)

            m, s, acc = lax.fori_loop(ilo, ihi, atom_body, (m, s, acc))

            @pl.when(r1 <= off + hi)
            def _():
                inv = 1.0 / (s + 1e-12)
                for j in range(NV):
                    outb[pl.ds(r * D + j * L, L)] = acc[j] * inv

            return (m, s, acc)

        return lax.fori_loop(sfirst, slast + 1, seg_body, carry)

    init = (nvec, zvec, tuple(zvec for _ in range(NV)))
    lax.fori_loop(0, nblk, blk_body, init)
    pltpu.sync_copy(
        outb, out_hbm.at[pl.ds(pl.multiple_of(w * (SPW * D), 8), SPW * D)])


def kernel(atom_features, atom_split, protSeq_features):
    ids = atom_split.astype(jnp.int32)
    bounds = jnp.arange(BP + 1, dtype=jnp.int32)
    rstarts = jnp.searchsorted(ids, bounds).astype(jnp.int32)
    rstarts = jnp.concatenate(
        [rstarts, jnp.full((RS_PAD - (BP + 1),), N, jnp.int32)])
    protp = jnp.concatenate(
        [protSeq_features,
         jnp.zeros((BP - B, D), jnp.float32)]).reshape(NW, SPW, D)
    protp = jnp.pad(protp, ((0, 0), (0, SPWP - SPW), (0, 0)))
    out = _wg(atom_features, ids, protp, rstarts)
    return out.reshape(BP, D)[:B]
